# Initial kernel scaffold; baseline (speedup 1.0000x reference)
import jax
import jax.numpy as jnp
from jax.experimental import pallas as pl


def kernel(prediction):
    raise NotImplementedError("write your pallas kernel here")


# speedup vs baseline: 102.1970x; 102.1970x over previous
import jax
import jax.numpy as jnp
from jax import lax
from jax.experimental import pallas as pl
from jax.experimental.pallas import tpu as pltpu

_NCLS = 80
_CONF = 0.05
_THR = 0.45
_NPAD = 5120
_CHUNK = 640
_K = 256
_NB = _NPAD // _K


def _prep_body(pred_ref, boxes_ref, ext_ref, maxc_ref):
    p = pred_ref[0]
    cx = p[:, 0:1]
    cy = p[:, 1:2]
    w = p[:, 2:3]
    h = p[:, 3:4]
    x1 = cx - w / 2.0
    y1 = cy - h / 2.0
    x2 = cx + w / 2.0
    y2 = cy + h / 2.0
    boxes_ref[0, :, 0:1] = x1
    boxes_ref[0, :, 1:2] = y1
    boxes_ref[0, :, 2:3] = x2
    boxes_ref[0, :, 3:4] = y2
    obj = p[:, 4:5]
    cl = p[:, 5:5 + _NCLS]
    cconf = jnp.max(cl, axis=1, keepdims=True)
    li = lax.broadcasted_iota(jnp.int32, cl.shape, 1)
    cpred = jnp.min(jnp.where(cl == cconf, li, _NCLS), axis=1, keepdims=True)
    score = obj * cconf
    mask = score >= _CONF
    ceff = jnp.where(mask, cpred, 127)
    key = ceff.astype(jnp.float32) * 4.0 - score
    ext_ref[0, :, 0:1] = obj
    ext_ref[0, :, 1:2] = cconf
    ext_ref[0, :, 2:3] = cpred.astype(jnp.float32)
    ext_ref[0, :, 3:4] = mask.astype(jnp.float32)
    ext_ref[0, :, 4:5] = ceff.astype(jnp.float32)
    ext_ref[0, :, 5:6] = key
    m = jnp.maximum(jnp.maximum(jnp.max(x1), jnp.max(y1)),
                    jnp.maximum(jnp.max(x2), jnp.max(y2)))
    c = pl.program_id(1)

    @pl.when(c == 0)
    def _():
        maxc_ref[0] = jnp.full((1, 128), m, jnp.float32)

    @pl.when(c > 0)
    def _():
        maxc_ref[0] = jnp.maximum(maxc_ref[0], m)


def _prep(pred):
    b = pred.shape[0]
    nchunks = _NPAD // _CHUNK
    return pl.pallas_call(
        _prep_body,
        grid=(b, nchunks),
        in_specs=[pl.BlockSpec((1, _CHUNK, 5 + _NCLS), lambda i, c: (i, c, 0))],
        out_specs=[
            pl.BlockSpec((1, _CHUNK, 4), lambda i, c: (i, c, 0)),
            pl.BlockSpec((1, _CHUNK, 8), lambda i, c: (i, c, 0)),
            pl.BlockSpec((1, 1, 128), lambda i, c: (i, 0, 0)),
        ],
        out_shape=[
            jax.ShapeDtypeStruct((b, _NPAD, 4), jnp.float32),
            jax.ShapeDtypeStruct((b, _NPAD, 8), jnp.float32),
            jax.ShapeDtypeStruct((b, 1, 128), jnp.float32),
        ],
    )(pred)


def _nms_body(srows_ref, scols_ref, maxb_ref, bcls_ref, keep_ref, sup_ref):
    mc = maxb_ref[0, 0, 0] + 1.0
    sup_ref[...] = jnp.zeros_like(sup_ref)
    upper = (lax.broadcasted_iota(jnp.int32, (_K, _K), 0)
             < lax.broadcasted_iota(jnp.int32, (_K, _K), 1))

    def cols(c0):
        off = scols_ref[0, 4:5, pl.ds(c0, _K)] * mc
        x1 = scols_ref[0, 0:1, pl.ds(c0, _K)] + off
        y1 = scols_ref[0, 1:2, pl.ds(c0, _K)] + off
        x2 = scols_ref[0, 2:3, pl.ds(c0, _K)] + off
        y2 = scols_ref[0, 3:4, pl.ds(c0, _K)] + off
        ar = jnp.maximum(x2 - x1, 0.0) * jnp.maximum(y2 - y1, 0.0)
        return x1, y1, x2, y2, ar

    def rows(r0):
        off = srows_ref[0, pl.ds(r0, _K), 4:5] * mc
        x1 = srows_ref[0, pl.ds(r0, _K), 0:1] + off
        y1 = srows_ref[0, pl.ds(r0, _K), 1:2] + off
        x2 = srows_ref[0, pl.ds(r0, _K), 2:3] + off
        y2 = srows_ref[0, pl.ds(r0, _K), 3:4] + off
        ar = jnp.maximum(x2 - x1, 0.0) * jnp.maximum(y2 - y1, 0.0)
        return x1, y1, x2, y2, ar

    def iou_gt(r, c):
        rx1, ry1, rx2, ry2, ra = r
        cx1, cy1, cx2, cy2, ca = c
        ltx = jnp.maximum(rx1, cx1)
        lty = jnp.maximum(ry1, cy1)
        rbx = jnp.minimum(rx2, cx2)
        rby = jnp.minimum(ry2, cy2)
        inter = jnp.maximum(rbx - ltx, 0.0) * jnp.maximum(rby - lty, 0.0)
        union = (ra + ca) - inter
        return (inter / (union + 1e-8)) > _THR

    def a_step(a, carry):
        r0 = a * _K
        ra = rows(r0)
        af = jnp.where(jnp.logical_and(iou_gt(ra, cols(r0)), upper), 1.0, 0.0)
        sup_in = sup_ref[:, pl.ds(r0, _K)]

        def w_cond(c):
            return jnp.logical_not(c[1])

        def w_body(c):
            s, _ = c
            cnt = lax.dot_general(1.0 - s, af, (((1,), (0,)), ((), ())),
                                  preferred_element_type=jnp.float32)
            ns = jnp.maximum(sup_in, (cnt > 0.5).astype(jnp.float32))
            return ns, jnp.all(ns == s)

        s_fin, _ = lax.while_loop(w_cond, w_body, (sup_in, jnp.asarray(False)))
        keep_a = 1.0 - s_fin
        keep_ref[0, :, pl.ds(r0, _K)] = keep_a
        end_a = bcls_ref[0, 1, a]

        def b_step(b, carry2):
            @pl.when(bcls_ref[0, 0, b] <= end_a)
            def _():
                ab = jnp.where(iou_gt(ra, cols(b * _K)), 1.0, 0.0)
                cnt = lax.dot_general(keep_a, ab, (((1,), (0,)), ((), ())),
                                      preferred_element_type=jnp.float32)
                cur = sup_ref[:, pl.ds(b * _K, _K)]
                sup_ref[:, pl.ds(b * _K, _K)] = jnp.maximum(
                    cur, (cnt > 0.5).astype(jnp.float32))
            return carry2

        lax.fori_loop(a + 1, _NB, b_step, 0)
        return carry

    lax.fori_loop(0, _NB, a_step, 0)


def _nms(srows, scols, maxb, bcls):
    b = srows.shape[0]
    return pl.pallas_call(
        _nms_body,
        grid=(b,),
        in_specs=[
            pl.BlockSpec((1, _NPAD, 6), lambda i: (i, 0, 0)),
            pl.BlockSpec((1, 6, _NPAD), lambda i: (i, 0, 0)),
            pl.BlockSpec((1, 1, 1), lambda i: (i, 0, 0),
                         memory_space=pltpu.SMEM),
            pl.BlockSpec((1, 2, _NB), lambda i: (i, 0, 0),
                         memory_space=pltpu.SMEM),
        ],
        out_specs=pl.BlockSpec((1, 8, _NPAD), lambda i: (i, 0, 0)),
        out_shape=jax.ShapeDtypeStruct((b, 8, _NPAD), jnp.float32),
        scratch_shapes=[pltpu.VMEM((8, _NPAD), jnp.float32)],
    )(srows, scols, maxb, bcls)


def kernel(prediction):
    b, n, _ = prediction.shape
    pred = jnp.pad(prediction, ((0, 0), (0, _NPAD - n), (0, 0)))
    boxes, ext, maxc = _prep(pred)
    key = ext[..., 5]
    perm = jnp.argsort(key, axis=-1)
    rowdata = jnp.concatenate([boxes, ext[..., 2:3], ext[..., 4:5]], axis=-1)
    srows = jnp.take_along_axis(rowdata, perm[..., None], axis=1)
    scols = srows.transpose(0, 2, 1)
    ceff_s = srows[..., 5].astype(jnp.int32)
    bcls = jnp.stack([ceff_s[:, ::_K], ceff_s[:, _K - 1::_K]], axis=1)
    maxb = maxc[:, 0:1, 0:1]
    keep8 = _nms(srows, scols, maxb, bcls)
    keep_sorted = keep8[:, 0, :]
    rows_idx = jnp.arange(b)[:, None]
    keep_orig = jnp.zeros((b, _NPAD), jnp.float32).at[rows_idx, perm].set(
        keep_sorted)
    keep = keep_orig[:, :n] * ext[:, :n, 3]
    dets = jnp.concatenate([boxes[:, :n, :], ext[:, :n, 0:3]], axis=-1)
    dets = dets * keep[..., None]
    return dets, keep
